# baseline (speedup 1.0000x reference)
import jax
import jax.numpy as jnp
from jax.experimental import pallas as pl
from jax.experimental.pallas import tpu as pltpu
from jax.experimental.pallas import tpu_sc as plsc

NUM_COLS = 26
VOCAB = 1000
D_MODEL = 64
BATCH = 4096
STRIDE = 1024
TOTAL = BATCH * NUM_COLS
WINDOW = 128
B_FMT = 128


def _combine_body(x_cat_ref, tables_ref, col_ref, comb_ref, idx_ref):
    comb_ref[: VOCAB + 1, :D_MODEL] = tables_ref[0] + col_ref[0]

    @pl.when(pl.program_id(0) == 0)
    def _():
        col_ids = jax.lax.broadcasted_iota(jnp.int32, (BATCH, NUM_COLS), 1)
        idx_ref[...] = x_cat_ref[...] + col_ids * STRIDE


_combine = pl.pallas_call(
    _combine_body,
    grid=(NUM_COLS,),
    in_specs=[
        pl.BlockSpec((BATCH, NUM_COLS), lambda c: (0, 0)),
        pl.BlockSpec((1, VOCAB + 1, D_MODEL), lambda c: (c, 0, 0)),
        pl.BlockSpec((1, 1, D_MODEL), lambda c: (c, 0, 0)),
    ],
    out_specs=[
        pl.BlockSpec((STRIDE, 128), lambda c: (c, 0)),
        pl.BlockSpec((BATCH, NUM_COLS), lambda c: (0, 0)),
    ],
    out_shape=[
        jax.ShapeDtypeStruct((NUM_COLS * STRIDE, 128), jnp.float32),
        jax.ShapeDtypeStruct((BATCH, NUM_COLS), jnp.int32),
    ],
)

_mesh = plsc.VectorSubcoreMesh(core_axis_name="c", subcore_axis_name="s")


@pl.kernel(
    out_type=jax.ShapeDtypeStruct((TOTAL, 128), jnp.float32),
    mesh=_mesh,
    compiler_params=pltpu.CompilerParams(use_tc_tiling_on_sc=True),
)
def _sc_gather(table_hbm, idx_hbm, out_hbm):
    def body(i_vmem, o_vmem):
        pltpu.sync_copy(table_hbm.at[i_vmem.at[0]], o_vmem)

    pltpu.emit_pipeline(
        body,
        grid=(TOTAL // WINDOW,),
        in_specs=[pl.BlockSpec((1, WINDOW), index_map=lambda i: (i, 0))],
        out_specs=[pl.BlockSpec((WINDOW, 128), index_map=lambda i: (i, 0))],
        core_axis_name=("c", "s"),
        dimension_semantics=(pltpu.PARALLEL,),
    )(idx_hbm, out_hbm)


def _format_body(gout_ref, out_ref):
    val = gout_ref[:, :D_MODEL]
    out_ref[...] = val.reshape(B_FMT, NUM_COLS, D_MODEL)


_format = pl.pallas_call(
    _format_body,
    grid=(BATCH // B_FMT,),
    in_specs=[pl.BlockSpec((B_FMT * NUM_COLS, 128), lambda i: (i, 0))],
    out_specs=pl.BlockSpec((B_FMT, NUM_COLS, D_MODEL), lambda i: (i, 0, 0)),
    out_shape=jax.ShapeDtypeStruct((BATCH, NUM_COLS, D_MODEL), jnp.float32),
)


def kernel(x_cat, tables, col_type):
    comb, idx = _combine(
        x_cat.astype(jnp.int32), tables, col_type.reshape(NUM_COLS, 1, D_MODEL)
    )
    gout = _sc_gather(comb, idx.reshape(TOTAL // 128, 128))
    return _format(gout)


# speedup vs baseline: 1.2670x; 1.2670x over previous
import jax
import jax.numpy as jnp
from jax.experimental import pallas as pl
from jax.experimental.pallas import tpu as pltpu
from jax.experimental.pallas import tpu_sc as plsc

NUM_COLS = 26
VOCAB = 1000
D_MODEL = 64
BATCH = 4096
STRIDE = 1024
TOTAL = BATCH * NUM_COLS
WINDOW = 128
CGRID = 2
CCOLS = NUM_COLS // CGRID


def _combine_body(x_cat_ref, tables_ref, col_ref, comb_ref, idx_ref):
    for k in range(CCOLS):
        comb_ref[k * STRIDE : k * STRIDE + VOCAB + 1, :] = (
            tables_ref[k] + col_ref[k]
        )

    @pl.when(pl.program_id(0) == 0)
    def _():
        col_ids = jax.lax.broadcasted_iota(jnp.int32, (BATCH, NUM_COLS), 1)
        idx_ref[...] = x_cat_ref[...] + col_ids * STRIDE


_combine = pl.pallas_call(
    _combine_body,
    grid=(CGRID,),
    in_specs=[
        pl.BlockSpec((BATCH, NUM_COLS), lambda c: (0, 0)),
        pl.BlockSpec((CCOLS, VOCAB + 1, D_MODEL), lambda c: (c, 0, 0)),
        pl.BlockSpec((CCOLS, 1, D_MODEL), lambda c: (c, 0, 0)),
    ],
    out_specs=[
        pl.BlockSpec((CCOLS * STRIDE, D_MODEL), lambda c: (c, 0)),
        pl.BlockSpec((BATCH, NUM_COLS), lambda c: (0, 0)),
    ],
    out_shape=[
        jax.ShapeDtypeStruct((NUM_COLS * STRIDE, D_MODEL), jnp.float32),
        jax.ShapeDtypeStruct((BATCH, NUM_COLS), jnp.int32),
    ],
)

_mesh = plsc.VectorSubcoreMesh(core_axis_name="c", subcore_axis_name="s")


@pl.kernel(
    out_type=jax.ShapeDtypeStruct((TOTAL, D_MODEL), jnp.float32),
    mesh=_mesh,
    compiler_params=pltpu.CompilerParams(use_tc_tiling_on_sc=False),
)
def _sc_gather(table_hbm, idx_hbm, out_hbm):
    def body(i_vmem, o_vmem):
        pltpu.sync_copy(table_hbm.at[i_vmem.at[0]], o_vmem)

    pltpu.emit_pipeline(
        body,
        grid=(TOTAL // WINDOW,),
        in_specs=[pl.BlockSpec((1, WINDOW), index_map=lambda i: (i, 0))],
        out_specs=[pl.BlockSpec((WINDOW, D_MODEL), index_map=lambda i: (i, 0))],
        core_axis_name=("c", "s"),
        dimension_semantics=(pltpu.PARALLEL,),
    )(idx_hbm, out_hbm)


def kernel(x_cat, tables, col_type):
    comb, idx = _combine(
        x_cat.astype(jnp.int32), tables, col_type.reshape(NUM_COLS, 1, D_MODEL)
    )
    flat = _sc_gather(comb, idx.reshape(TOTAL // 128, 128))
    return flat.reshape(BATCH, NUM_COLS, D_MODEL)


# speedup vs baseline: 1.2778x; 1.0085x over previous
import jax
import jax.numpy as jnp
from jax.experimental import pallas as pl
from jax.experimental.pallas import tpu as pltpu
from jax.experimental.pallas import tpu_sc as plsc

NUM_COLS = 26
VOCAB = 1000
D_MODEL = 64
BATCH = 4096
STRIDE = 1024
TOTAL = BATCH * NUM_COLS
WINDOW = 128
CGRID = 2
CCOLS = NUM_COLS // CGRID


def _combine_body(x_cat_ref, tables_ref, col_ref, comb_ref, idx_ref):
    for k in range(CCOLS):
        comb_ref[k * STRIDE : k * STRIDE + VOCAB + 1, :] = (
            tables_ref[k] + col_ref[k]
        )

    @pl.when(pl.program_id(0) == 0)
    def _():
        col_ids = jax.lax.broadcasted_iota(jnp.int32, (BATCH, NUM_COLS), 1)
        idx_ref[...] = x_cat_ref[...] + col_ids * STRIDE


_combine = pl.pallas_call(
    _combine_body,
    grid=(CGRID,),
    in_specs=[
        pl.BlockSpec((BATCH, NUM_COLS), lambda c: (0, 0)),
        pl.BlockSpec((CCOLS, VOCAB + 1, D_MODEL), lambda c: (c, 0, 0)),
        pl.BlockSpec((CCOLS, 1, D_MODEL), lambda c: (c, 0, 0)),
    ],
    out_specs=[
        pl.BlockSpec((CCOLS * STRIDE, D_MODEL), lambda c: (c, 0)),
        pl.BlockSpec((BATCH, NUM_COLS), lambda c: (0, 0)),
    ],
    out_shape=[
        jax.ShapeDtypeStruct((NUM_COLS * STRIDE, D_MODEL), jnp.float32),
        jax.ShapeDtypeStruct((BATCH, NUM_COLS), jnp.int32),
    ],
)

_mesh = plsc.VectorSubcoreMesh(core_axis_name="c", subcore_axis_name="s")


@pl.kernel(
    out_type=jax.ShapeDtypeStruct((TOTAL, D_MODEL), jnp.float32),
    mesh=_mesh,
    compiler_params=pltpu.CompilerParams(use_tc_tiling_on_sc=False),
)
def _sc_gather(table_hbm, idx_hbm, out_hbm):
    def body(i_vmem, o_vmem):
        pltpu.sync_copy(table_hbm.at[i_vmem.at[0]], o_vmem.at[pl.ds(0, WINDOW)])
        pltpu.sync_copy(
            table_hbm.at[i_vmem.at[1]], o_vmem.at[pl.ds(WINDOW, WINDOW)]
        )

    pltpu.emit_pipeline(
        body,
        grid=(TOTAL // (2 * WINDOW),),
        in_specs=[pl.BlockSpec((2, WINDOW), index_map=lambda i: (i, 0))],
        out_specs=[
            pl.BlockSpec((2 * WINDOW, D_MODEL), index_map=lambda i: (i, 0))
        ],
        core_axis_name=("c", "s"),
        dimension_semantics=(pltpu.PARALLEL,),
    )(idx_hbm, out_hbm)


def kernel(x_cat, tables, col_type):
    comb, idx = _combine(
        x_cat.astype(jnp.int32), tables, col_type.reshape(NUM_COLS, 1, D_MODEL)
    )
    flat = _sc_gather(comb, idx.reshape(TOTAL // 128, 128))
    return flat.reshape(BATCH, NUM_COLS, D_MODEL)


# speedup vs baseline: 1.3271x; 1.0386x over previous
import jax
import jax.numpy as jnp
from jax.experimental import pallas as pl
from jax.experimental.pallas import tpu as pltpu
from jax.experimental.pallas import tpu_sc as plsc

NUM_COLS = 26
VOCAB = 1000
D_MODEL = 64
BATCH = 4096
STRIDE = 1024
TOTAL = BATCH * NUM_COLS
WINDOW = 128
CGRID = 2
CCOLS = NUM_COLS // CGRID


def _combine_body(x_cat_ref, tables_ref, col_ref, comb_ref, idx_ref):
    for k in range(CCOLS):
        comb_ref[k * STRIDE : k * STRIDE + VOCAB + 1, :] = (
            tables_ref[k] + col_ref[k]
        )

    @pl.when(pl.program_id(0) == 0)
    def _():
        col_ids = jax.lax.broadcasted_iota(jnp.int32, (BATCH, NUM_COLS), 1)
        idx_ref[...] = x_cat_ref[...] + col_ids * STRIDE


_combine = pl.pallas_call(
    _combine_body,
    grid=(CGRID,),
    in_specs=[
        pl.BlockSpec((BATCH, NUM_COLS), lambda c: (0, 0)),
        pl.BlockSpec((CCOLS, VOCAB + 1, D_MODEL), lambda c: (c, 0, 0)),
        pl.BlockSpec((CCOLS, 1, D_MODEL), lambda c: (c, 0, 0)),
    ],
    out_specs=[
        pl.BlockSpec((CCOLS * STRIDE, D_MODEL), lambda c: (c, 0)),
        pl.BlockSpec((BATCH, NUM_COLS), lambda c: (0, 0)),
    ],
    out_shape=[
        jax.ShapeDtypeStruct((NUM_COLS * STRIDE, D_MODEL), jnp.float32),
        jax.ShapeDtypeStruct((BATCH, NUM_COLS), jnp.int32),
    ],
)

_mesh = plsc.VectorSubcoreMesh(core_axis_name="c", subcore_axis_name="s")


@pl.kernel(
    out_type=jax.ShapeDtypeStruct((TOTAL, D_MODEL), jnp.float32),
    mesh=_mesh,
    compiler_params=pltpu.CompilerParams(use_tc_tiling_on_sc=False),
    scratch_types=[pltpu.SemaphoreType.DMA],
)
def _sc_gather(table_hbm, idx_hbm, out_hbm, sem):
    def body(i_vmem, o_vmem):
        c0 = pltpu.make_async_copy(
            table_hbm.at[i_vmem.at[0]], o_vmem.at[pl.ds(0, WINDOW)], sem
        )
        c1 = pltpu.make_async_copy(
            table_hbm.at[i_vmem.at[1]], o_vmem.at[pl.ds(WINDOW, WINDOW)], sem
        )
        c0.start()
        c1.start()
        c0.wait()
        c1.wait()

    pltpu.emit_pipeline(
        body,
        grid=(TOTAL // (2 * WINDOW),),
        in_specs=[pl.BlockSpec((2, WINDOW), index_map=lambda i: (i, 0))],
        out_specs=[
            pl.BlockSpec((2 * WINDOW, D_MODEL), index_map=lambda i: (i, 0))
        ],
        core_axis_name=("c", "s"),
        dimension_semantics=(pltpu.PARALLEL,),
    )(idx_hbm, out_hbm)


def kernel(x_cat, tables, col_type):
    comb, idx = _combine(
        x_cat.astype(jnp.int32), tables, col_type.reshape(NUM_COLS, 1, D_MODEL)
    )
    flat = _sc_gather(comb, idx.reshape(TOTAL // 128, 128))
    return flat.reshape(BATCH, NUM_COLS, D_MODEL)


# speedup vs baseline: 1.3455x; 1.0139x over previous
import jax
import jax.numpy as jnp
from jax.experimental import pallas as pl
from jax.experimental.pallas import tpu as pltpu
from jax.experimental.pallas import tpu_sc as plsc

NUM_COLS = 26
VOCAB = 1000
D_MODEL = 64
BATCH = 4096
STRIDE = 1024
TOTAL = BATCH * NUM_COLS
WINDOW = 128
CGRID = 2
CCOLS = NUM_COLS // CGRID


def _combine_body(x_cat_ref, tables_ref, col_ref, comb_ref, idx_ref):
    for k in range(CCOLS):
        comb_ref[k * STRIDE : k * STRIDE + VOCAB + 1, :] = (
            tables_ref[k] + col_ref[k]
        )

    @pl.when(pl.program_id(0) == 0)
    def _():
        col_ids = jax.lax.broadcasted_iota(jnp.int32, (BATCH, NUM_COLS), 1)
        idx_ref[...] = x_cat_ref[...] + col_ids * STRIDE


_combine = pl.pallas_call(
    _combine_body,
    grid=(CGRID,),
    in_specs=[
        pl.BlockSpec((BATCH, NUM_COLS), lambda c: (0, 0)),
        pl.BlockSpec((CCOLS, VOCAB + 1, D_MODEL), lambda c: (c, 0, 0)),
        pl.BlockSpec((CCOLS, 1, D_MODEL), lambda c: (c, 0, 0)),
    ],
    out_specs=[
        pl.BlockSpec((CCOLS * STRIDE, D_MODEL), lambda c: (c, 0)),
        pl.BlockSpec((BATCH, NUM_COLS), lambda c: (0, 0)),
    ],
    out_shape=[
        jax.ShapeDtypeStruct((NUM_COLS * STRIDE, D_MODEL), jnp.float32),
        jax.ShapeDtypeStruct((BATCH, NUM_COLS), jnp.int32),
    ],
)

_mesh = plsc.VectorSubcoreMesh(core_axis_name="c", subcore_axis_name="s")

NSTREAM = 4


@pl.kernel(
    out_type=jax.ShapeDtypeStruct((TOTAL, D_MODEL), jnp.float32),
    mesh=_mesh,
    compiler_params=pltpu.CompilerParams(use_tc_tiling_on_sc=False),
    scratch_types=[pltpu.SemaphoreType.DMA],
)
def _sc_gather(table_hbm, idx_hbm, out_hbm, sem):
    def body(i_vmem, o_vmem):
        copies = [
            pltpu.make_async_copy(
                table_hbm.at[i_vmem.at[k]],
                o_vmem.at[pl.ds(k * WINDOW, WINDOW)],
                sem,
            )
            for k in range(NSTREAM)
        ]
        for c in copies:
            c.start()
        for c in copies:
            c.wait()

    pltpu.emit_pipeline(
        body,
        grid=(TOTAL // (NSTREAM * WINDOW),),
        in_specs=[pl.BlockSpec((NSTREAM, WINDOW), index_map=lambda i: (i, 0))],
        out_specs=[
            pl.BlockSpec((NSTREAM * WINDOW, D_MODEL), index_map=lambda i: (i, 0))
        ],
        core_axis_name=("c", "s"),
        dimension_semantics=(pltpu.PARALLEL,),
    )(idx_hbm, out_hbm)


def kernel(x_cat, tables, col_type):
    comb, idx = _combine(
        x_cat.astype(jnp.int32), tables, col_type.reshape(NUM_COLS, 1, D_MODEL)
    )
    flat = _sc_gather(comb, idx.reshape(TOTAL // 128, 128))
    return flat.reshape(BATCH, NUM_COLS, D_MODEL)


# speedup vs baseline: 1.3968x; 1.0382x over previous
import jax
import jax.numpy as jnp
from jax.experimental import pallas as pl
from jax.experimental.pallas import tpu as pltpu
from jax.experimental.pallas import tpu_sc as plsc

NUM_COLS = 26
VOCAB = 1000
D_MODEL = 64
BATCH = 4096
STRIDE = 1024
TOTAL = BATCH * NUM_COLS
WINDOW = 128
CGRID = 2
CCOLS = NUM_COLS // CGRID


def _combine_body(tables_ref, col_ref, comb_ref):
    for k in range(CCOLS):
        comb_ref[k * STRIDE : k * STRIDE + VOCAB + 1, :] = (
            tables_ref[k] + col_ref[k]
        )


_combine = pl.pallas_call(
    _combine_body,
    grid=(CGRID,),
    in_specs=[
        pl.BlockSpec((CCOLS, VOCAB + 1, D_MODEL), lambda c: (c, 0, 0)),
        pl.BlockSpec((CCOLS, 1, D_MODEL), lambda c: (c, 0, 0)),
    ],
    out_specs=pl.BlockSpec((CCOLS * STRIDE, D_MODEL), lambda c: (c, 0)),
    out_shape=jax.ShapeDtypeStruct((NUM_COLS * STRIDE, D_MODEL), jnp.float32),
)

_mesh = plsc.VectorSubcoreMesh(core_axis_name="c", subcore_axis_name="s")

BROWS = 32
WROWS = BROWS * NUM_COLS
NCHUNK = WROWS // 16
GSLICE = 104
LANES = 16


@pl.kernel(
    out_type=jax.ShapeDtypeStruct((TOTAL, D_MODEL), jnp.float32),
    mesh=_mesh,
    compiler_params=pltpu.CompilerParams(
        use_tc_tiling_on_sc=False, needs_layout_passes=False
    ),
    scratch_types=[
        pltpu.VMEM((WROWS,), jnp.int32),
        pltpu.VMEM((WROWS,), jnp.int32),
        pltpu.VMEM((WROWS,), jnp.int32),
        pltpu.SemaphoreType.DMA,
    ],
)
def _sc_gather(x_hbm, table_hbm, out_hbm, rows_v, cols_v, idx_v, sem):
    @pl.loop(0, WROWS, step=LANES)
    def _(j):
        p = jax.lax.iota(jnp.int32, LANES) + j
        rows_v[pl.ds(j, LANES)] = p // NUM_COLS
        cols_v[pl.ds(j, LANES)] = jax.lax.rem(p, NUM_COLS)

    def body(x_vmem, o_vmem):
        @pl.loop(0, WROWS, step=LANES)
        def _(j):
            r = rows_v[pl.ds(j, LANES)]
            c = cols_v[pl.ds(j, LANES)]
            xv = plsc.load_gather(x_vmem, [r, c])
            idx_v[pl.ds(j, LANES)] = xv + c * STRIDE

        copies = [
            pltpu.make_async_copy(
                table_hbm.at[idx_v.at[pl.ds(k * GSLICE, GSLICE)]],
                o_vmem.at[pl.ds(k * GSLICE, GSLICE)],
                sem,
            )
            for k in range(WROWS // GSLICE)
        ]
        for c in copies:
            c.start()
        for c in copies:
            c.wait()

    pltpu.emit_pipeline(
        body,
        grid=(BATCH // BROWS,),
        in_specs=[pl.BlockSpec((BROWS, NUM_COLS), index_map=lambda i: (i, 0))],
        out_specs=[pl.BlockSpec((WROWS, D_MODEL), index_map=lambda i: (i, 0))],
        core_axis_name=("c", "s"),
        dimension_semantics=(pltpu.PARALLEL,),
    )(x_hbm, out_hbm)


def kernel(x_cat, tables, col_type):
    comb = _combine(tables, col_type.reshape(NUM_COLS, 1, D_MODEL))
    flat = _sc_gather(x_cat.astype(jnp.int32), comb)
    return flat.reshape(BATCH, NUM_COLS, D_MODEL)
